# baseline (speedup 1.0000x reference)
import functools

import jax
import jax.numpy as jnp
from jax import lax
from jax.experimental import pallas as pl
from jax.experimental.pallas import tpu as pltpu
from jax.experimental.pallas import tpu_sc as plsc

NC = 2
NS = 16
L = 16
NW = NC * NS

ROWS = 16384
COLS = 200
JT = COLS // 8
IHI = ROWS // 128
SLABS = JT * IHI
NB = 5
NT = 5

_mesh = plsc.VectorSubcoreMesh(core_axis_name="c", subcore_axis_name="s")


@functools.partial(
    pl.kernel,
    mesh=_mesh,
    out_type=jax.ShapeDtypeStruct((SLABS, 8, 128), jnp.float32),
    scratch_types=[
        pltpu.VMEM((2, L), jnp.float32),
        pltpu.VMEM((NB, 4, 8, 128), jnp.int32),
        pltpu.VMEM((NB, 4, 8, 128), jnp.float32),
        [pltpu.SemaphoreType.DMA] * (2 * NB),
    ],
)
def _emb_lookup(idx_hbm, w_hbm, out_hbm, w_v, idx_v, out_v, sems):
    wid = lax.axis_index("s") * NC + lax.axis_index("c")
    u0 = wid * NB * NT
    s_in, s_out = sems[:NB], sems[NB:]

    pltpu.sync_copy(w_hbm, w_v)
    w0 = w_v[0, :]
    d = w_v[1, :] - w0

    def in_copy(u, m):
        return pltpu.make_async_copy(
            idx_hbm.at[pl.ds(u * 4, 4), :, :], idx_v.at[m], s_in[m])

    def out_copy(u, m):
        return pltpu.make_async_copy(
            out_v.at[m], out_hbm.at[pl.ds(u * 4, 4), :, :], s_out[m])

    def round_body(t, _):
        for m in range(NB):
            in_copy(u0 + t * NB + m, m).start()
        for m in range(NB):
            u = u0 + t * NB + m
            in_copy(u, m).wait()

            @pl.when(t > 0)
            def _():
                out_copy(u - NB, m).wait()

            def _unit(r, carry):
                for jj in range(8):
                    for off in range(0, 128, L):
                        x = idx_v[m, r, jj, pl.ds(off, L)]
                        out_v[m, r, jj, pl.ds(off, L)] = (
                            w0 + d * x.astype(jnp.float32))
                return carry

            lax.fori_loop(0, 4, _unit, 0)

            out_copy(u, m).start()
        return 0

    lax.fori_loop(0, NT, round_body, 0)
    for m in range(NB):
        out_copy(u0 + (NT - 1) * NB + m, m).wait()


def kernel(idx, weight):
    idx3 = (idx.T.reshape(JT, 8, IHI, 128)
            .transpose(0, 2, 1, 3).reshape(SLABS, 8, 128))
    wb = jnp.broadcast_to(weight.astype(jnp.float32), (2, L))
    out3 = _emb_lookup(idx3, wb)
    return (out3.reshape(JT, IHI, 8, 128).transpose(0, 2, 1, 3)
            .reshape(COLS, ROWS).T.reshape(ROWS, COLS, 1))
